# Initial kernel scaffold; baseline (speedup 1.0000x reference)
import jax
import jax.numpy as jnp
from jax.experimental import pallas as pl


def kernel(E0, E1, E2, E3, E4, E5, E6, E7, R0, R1, R2, R3, R4, R5, R6, R7, e1_idx, rel_idx):
    raise NotImplementedError("write your pallas kernel here")


# speedup vs baseline: 1.6143x; 1.6143x over previous
import functools

import jax
import jax.numpy as jnp
from jax import lax
from jax.experimental import pallas as pl
from jax.experimental.pallas import tpu as pltpu

NUM_ENT = 100000
DIM = 32
B = 1024
TN = 2048
NBLK = (NUM_ENT + TN - 1) // TN


def _octonion_mul(O1, O2):
    x0, x1, x2, x3, x4, x5, x6, x7 = O1
    y0, y1, y2, y3, y4, y5, y6, y7 = O2
    e0 = x0*y0 - x1*y1 - x2*y2 - x3*y3 - x4*y4 - x5*y5 - x6*y6 - x7*y7
    e1 = x0*y1 + x1*y0 + x2*y3 - x3*y2 + x4*y5 - x5*y4 - x6*y7 + x7*y6
    e2 = x0*y2 - x1*y3 + x2*y0 + x3*y1 + x4*y6 + x5*y7 - x6*y4 - x7*y5
    e3 = x0*y3 + x1*y2 - x2*y1 + x3*y0 + x4*y7 - x5*y6 + x6*y5 - x7*y4
    e4 = x0*y4 - x1*y5 - x2*y6 - x3*y7 + x4*y0 + x5*y1 + x6*y2 + x7*y3
    e5 = x0*y5 + x1*y4 - x2*y7 + x3*y6 - x4*y1 + x5*y0 - x6*y3 + x7*y2
    e6 = x0*y6 + x1*y7 + x2*y4 - x3*y5 - x4*y2 + x5*y3 + x6*y0 - x7*y1
    e7 = x0*y7 - x1*y6 + x2*y5 + x3*y4 - x4*y3 - x5*y2 + x6*y1 + x7*y0
    return (e0, e1, e2, e3, e4, e5, e6, e7)


def _score_kernel(*refs):
    hrefs = refs[0:8]
    rrefs = refs[8:16]
    erefs = refs[16:24]
    out_ref = refs[24]
    hs = refs[25]

    @pl.when(pl.program_id(0) == 0)
    def _build_h():
        ys = [r[...] for r in rrefs]
        inv = lax.rsqrt(sum(y * y for y in ys))
        ys = [y * inv for y in ys]
        xs = [h[...] for h in hrefs]
        es = _octonion_mul(xs, ys)
        for i in range(8):
            hs[i] = es[i]

    acc = jnp.zeros((B, TN), jnp.float32)
    for i in range(8):
        acc = acc + lax.dot_general(
            hs[i], erefs[i][...],
            (((1,), (1,)), ((), ())),
            preferred_element_type=jnp.float32)
    out_ref[...] = jax.nn.sigmoid(acc)


def _score(heads, rels, ents, interpret=False):
    full = pl.BlockSpec((B, DIM), lambda n: (0, 0))
    eblk = pl.BlockSpec((TN, DIM), lambda n: (n, 0))
    return pl.pallas_call(
        _score_kernel,
        grid=(NBLK,),
        in_specs=[full] * 16 + [eblk] * 8,
        out_specs=pl.BlockSpec((B, TN), lambda n: (0, n)),
        out_shape=jax.ShapeDtypeStruct((B, NUM_ENT), jnp.float32),
        scratch_shapes=[pltpu.VMEM((8, B, DIM), jnp.float32)],
        compiler_params=pltpu.CompilerParams(
            dimension_semantics=("arbitrary",)),
        interpret=interpret,
    )(*heads, *rels, *ents)


def kernel(E0, E1, E2, E3, E4, E5, E6, E7,
           R0, R1, R2, R3, R4, R5, R6, R7, e1_idx, rel_idx):
    ents = (E0, E1, E2, E3, E4, E5, E6, E7)
    rel_tables = (R0, R1, R2, R3, R4, R5, R6, R7)
    heads = tuple(jnp.take(E, e1_idx, axis=0) for E in ents)
    rels = tuple(jnp.take(R, rel_idx, axis=0) for R in rel_tables)
    return _score(heads, rels, ents)
